# Initial kernel scaffold; baseline (speedup 1.0000x reference)
import jax
import jax.numpy as jnp
from jax.experimental import pallas as pl


def kernel(Z, edge_index, rbf, d, emb_table, W_rbf, b_rbf, W_dense, b_dense):
    raise NotImplementedError("write your pallas kernel here")


# speedup vs baseline: 1.2548x; 1.2548x over previous
import functools

import numpy as np
import jax
import jax.numpy as jnp
from jax import lax
from jax.experimental import pallas as pl
from jax.experimental.pallas import tpu as pltpu
from jax.experimental.pallas import tpu_sc as plsc

_EMB = 128
_R = 6
_CUTOFF = 5.0
_N = 10000
_E = 320000

_NW = 32
_EPW = _E // _NW
_NPAD = 10240
_HPW = _NPAD // _NW
_HCH = 80

_BE = 4000
_GRID = _E // _BE


def _sc_gather(src, dst, Z, Zpad, emb_table):
    mesh = plsc.VectorSubcoreMesh(core_axis_name="c", subcore_axis_name="s")

    @functools.partial(
        pl.kernel,
        mesh=mesh,
        compiler_params=pltpu.CompilerParams(needs_layout_passes=False),
        out_type=(
            jax.ShapeDtypeStruct((_E,), jnp.int32),
            jax.ShapeDtypeStruct((_E,), jnp.int32),
            jax.ShapeDtypeStruct((_NPAD, _EMB), jnp.float32),
        ),
        scratch_types=[
            pltpu.VMEM((_N,), jnp.int32),
            pltpu.VMEM((_EPW,), jnp.int32),
            pltpu.VMEM((_EPW,), jnp.int32),
            pltpu.VMEM((_HPW,), jnp.int32),
            pltpu.VMEM((_HCH, _EMB), jnp.float32),
            pltpu.SemaphoreType.DMA,
        ],
    )
    def sc_kern(src_hbm, dst_hbm, z_hbm, zpad_hbm, emb_hbm,
                zsrc_hbm, zdst_hbm, h_hbm,
                z_v, idx_v, out_v, hidx_v, hrow_v, sem):
        wid = lax.axis_index("s") * 2 + lax.axis_index("c")
        ebase = wid * _EPW
        pltpu.sync_copy(z_hbm, z_v)

        def gather_endpoint(ep_hbm, out_hbm):
            pltpu.sync_copy(ep_hbm.at[pl.ds(ebase, _EPW)], idx_v)

            def body(i, carry):
                ids = idx_v[pl.ds(i * 16, 16)]
                out_v[pl.ds(i * 16, 16)] = plsc.load_gather(z_v, [ids])
                return carry

            lax.fori_loop(0, _EPW // 16, body, 0)
            pltpu.sync_copy(out_v, out_hbm.at[pl.ds(ebase, _EPW)])

        gather_endpoint(src_hbm, zsrc_hbm)
        gather_endpoint(dst_hbm, zdst_hbm)

        hbase = wid * _HPW
        pltpu.sync_copy(zpad_hbm.at[pl.ds(hbase, _HPW)], hidx_v)
        for j in range(_HPW // _HCH):
            pltpu.async_copy(
                emb_hbm.at[hidx_v.at[pl.ds(j * _HCH, _HCH)]], hrow_v, sem
            ).wait()
            pltpu.sync_copy(hrow_v, h_hbm.at[pl.ds(hbase + j * _HCH, _HCH), :])

    return sc_kern(src, dst, Z, Zpad, emb_table)


def _tc_body(zsrc_ref, zdst_ref, rbf_ref, d_ref,
             emb_ref, wd_ref, wrbf_ref, brbf_ref, bdense_ref,
             m_ref, env_ref, t1_s, t2_s, w3p_s, bias_s):
    @pl.when(pl.program_id(0) == 0)
    def _init():
        emb = emb_ref[...]
        w3 = wd_ref[2 * _EMB:3 * _EMB, :]
        t1_s[...] = jnp.dot(emb, wd_ref[0:_EMB, :],
                            preferred_element_type=jnp.float32)
        t2_s[...] = jnp.dot(emb, wd_ref[_EMB:2 * _EMB, :],
                            preferred_element_type=jnp.float32)
        w3p_s[...] = jnp.dot(wrbf_ref[...], w3,
                             preferred_element_type=jnp.float32)
        bias_s[...] = jnp.dot(brbf_ref[...], w3,
                              preferred_element_type=jnp.float32) + bdense_ref[...]

    zs = zsrc_ref[0]
    zd = zdst_ref[0]
    lane = lax.broadcasted_iota(jnp.int32, (_BE, _EMB), 1)
    oh_s = (zs == lane).astype(jnp.float32)
    oh_d = (zd == lane).astype(jnp.float32)
    acc = jnp.dot(oh_s, t1_s[...], preferred_element_type=jnp.float32)
    acc = acc + jnp.dot(oh_d, t2_s[...], preferred_element_type=jnp.float32)
    acc = acc + jnp.dot(rbf_ref[0], w3p_s[...],
                        preferred_element_type=jnp.float32)
    m_ref[0] = acc + bias_s[...]

    x = d_ref[0] * (1.0 / _CUTOFF)
    inv = 1.0 / x
    x2 = x * x
    x5 = x2 * x2 * x
    envl = inv + x5 * (-28.0 + x * (48.0 - 21.0 * x))
    nvals = (lax.broadcasted_iota(jnp.int32, (_BE, _R), 1) + 1).astype(jnp.float32)
    s = jnp.sin(nvals * np.float32(np.pi) * x)
    env_ref[0] = (envl * inv * np.float32(np.sqrt(2.0 / _CUTOFF))) * s


def kernel(Z, edge_index, rbf, d, emb_table, W_rbf, b_rbf, W_dense, b_dense):
    Zpad = jnp.pad(Z.astype(jnp.int32), (0, _NPAD - _N))
    ei = edge_index.astype(jnp.int32)
    zsrc, zdst, hpad = _sc_gather(ei[0], ei[1],
                                  Z.astype(jnp.int32), Zpad, emb_table)

    zsrc3 = zsrc.reshape(_GRID, _BE, 1)
    zdst3 = zdst.reshape(_GRID, _BE, 1)
    rbf3 = jnp.pad(rbf, ((0, 0), (0, 8 - _R))).reshape(_GRID, _BE, 8)
    d3 = d.reshape(_GRID, _BE, 1)
    emb_pad = jnp.pad(emb_table, ((0, _EMB - emb_table.shape[0]), (0, 0)))
    wrbf_pad = jnp.pad(W_rbf, ((0, 8 - _R), (0, 0)))
    brbf2 = b_rbf.reshape(1, _EMB)
    bdense2 = b_dense.reshape(1, _EMB)

    m3, env3 = pl.pallas_call(
        _tc_body,
        grid=(_GRID,),
        in_specs=[
            pl.BlockSpec((1, _BE, 1), lambda i: (i, 0, 0)),
            pl.BlockSpec((1, _BE, 1), lambda i: (i, 0, 0)),
            pl.BlockSpec((1, _BE, 8), lambda i: (i, 0, 0)),
            pl.BlockSpec((1, _BE, 1), lambda i: (i, 0, 0)),
            pl.BlockSpec((_EMB, _EMB), lambda i: (0, 0)),
            pl.BlockSpec((3 * _EMB, _EMB), lambda i: (0, 0)),
            pl.BlockSpec((8, _EMB), lambda i: (0, 0)),
            pl.BlockSpec((1, _EMB), lambda i: (0, 0)),
            pl.BlockSpec((1, _EMB), lambda i: (0, 0)),
        ],
        out_specs=[
            pl.BlockSpec((1, _BE, _EMB), lambda i: (i, 0, 0)),
            pl.BlockSpec((1, _BE, _R), lambda i: (i, 0, 0)),
        ],
        out_shape=[
            jax.ShapeDtypeStruct((_GRID, _BE, _EMB), jnp.float32),
            jax.ShapeDtypeStruct((_GRID, _BE, _R), jnp.float32),
        ],
        scratch_shapes=[
            pltpu.VMEM((_EMB, _EMB), jnp.float32),
            pltpu.VMEM((_EMB, _EMB), jnp.float32),
            pltpu.VMEM((8, _EMB), jnp.float32),
            pltpu.VMEM((1, _EMB), jnp.float32),
        ],
    )(zsrc3, zdst3, rbf3, d3, emb_pad, W_dense, wrbf_pad, brbf2, bdense2)

    h = hpad[:_N]
    m = m3.reshape(_E, _EMB)
    rbf_env = env3.reshape(_E, _R)
    return (h, m, rbf_env)


# speedup vs baseline: 2.3856x; 1.9012x over previous
import functools

import numpy as np
import jax
import jax.numpy as jnp
from jax import lax
from jax.experimental import pallas as pl
from jax.experimental.pallas import tpu as pltpu
from jax.experimental.pallas import tpu_sc as plsc

_EMB = 128
_R = 6
_CUTOFF = 5.0
_N = 10000
_E = 320000

_NW = 32
_EPW = _E // _NW
_NPAD = 10240
_HPW = _NPAD // _NW
_HCH = 80

_BE = 3200
_GRID = _E // _BE
_DSUB = _BE // 128


def _sc_gather(src, dst, Z, Zpad, emb_table):
    mesh = plsc.VectorSubcoreMesh(core_axis_name="c", subcore_axis_name="s")

    @functools.partial(
        pl.kernel,
        mesh=mesh,
        compiler_params=pltpu.CompilerParams(needs_layout_passes=False),
        out_type=(
            jax.ShapeDtypeStruct((_E,), jnp.int32),
            jax.ShapeDtypeStruct((_E,), jnp.int32),
            jax.ShapeDtypeStruct((_NPAD, _EMB), jnp.float32),
        ),
        scratch_types=[
            pltpu.VMEM((_N,), jnp.int32),
            pltpu.VMEM((_EPW,), jnp.int32),
            pltpu.VMEM((_EPW,), jnp.int32),
            pltpu.VMEM((_HPW,), jnp.int32),
            pltpu.VMEM((_HCH, _EMB), jnp.float32),
            pltpu.SemaphoreType.DMA,
        ],
    )
    def sc_kern(src_hbm, dst_hbm, z_hbm, zpad_hbm, emb_hbm,
                zsrc_hbm, zdst_hbm, h_hbm,
                z_v, idx_v, out_v, hidx_v, hrow_v, sem):
        wid = lax.axis_index("s") * 2 + lax.axis_index("c")
        ebase = wid * _EPW
        pltpu.sync_copy(z_hbm, z_v)

        def gather_endpoint(ep_hbm, out_hbm):
            pltpu.sync_copy(ep_hbm.at[pl.ds(ebase, _EPW)], idx_v)

            def body(i, carry):
                ids = idx_v[pl.ds(i * 16, 16)]
                out_v[pl.ds(i * 16, 16)] = plsc.load_gather(z_v, [ids])
                return carry

            lax.fori_loop(0, _EPW // 16, body, 0)
            pltpu.sync_copy(out_v, out_hbm.at[pl.ds(ebase, _EPW)])

        gather_endpoint(src_hbm, zsrc_hbm)
        gather_endpoint(dst_hbm, zdst_hbm)

        hbase = wid * _HPW
        pltpu.sync_copy(zpad_hbm.at[pl.ds(hbase, _HPW)], hidx_v)
        for j in range(_HPW // _HCH):
            pltpu.async_copy(
                emb_hbm.at[hidx_v.at[pl.ds(j * _HCH, _HCH)]], hrow_v, sem
            ).wait()
            pltpu.sync_copy(hrow_v, h_hbm.at[pl.ds(hbase + j * _HCH, _HCH), :])

    return sc_kern(src, dst, Z, Zpad, emb_table)


def _tc_body(zsrc_ref, zdst_ref, rbf_ref, d_ref,
             emb_ref, wd_ref, wrbf_ref, brbf_ref, bdense_ref,
             m_ref, env_ref, t1_s, t2_s, w3p_s, bias_s):
    @pl.when(pl.program_id(0) == 0)
    def _init():
        emb = emb_ref[...]
        w3 = wd_ref[2 * _EMB:3 * _EMB, :]
        t1_s[...] = jnp.dot(emb, wd_ref[0:_EMB, :],
                            preferred_element_type=jnp.float32)
        t2_s[...] = jnp.dot(emb, wd_ref[_EMB:2 * _EMB, :],
                            preferred_element_type=jnp.float32)
        w3p_s[...] = jnp.dot(wrbf_ref[...], w3,
                             preferred_element_type=jnp.float32)
        bias_s[...] = jnp.dot(brbf_ref[...], w3,
                              preferred_element_type=jnp.float32) + bdense_ref[...]

    zs = zsrc_ref[0]
    zd = zdst_ref[0]
    lane = lax.broadcasted_iota(jnp.int32, (_BE, _EMB), 1)
    oh_s = (zs == lane).astype(jnp.float32)
    oh_d = (zd == lane).astype(jnp.float32)
    acc = jnp.dot(oh_s, t1_s[...], preferred_element_type=jnp.float32)
    acc = acc + jnp.dot(oh_d, t2_s[...], preferred_element_type=jnp.float32)
    acc = acc + jnp.dot(rbf_ref[0], w3p_s[...],
                        preferred_element_type=jnp.float32)
    m_ref[0] = acc + bias_s[...]

    x = d_ref[0] * (1.0 / _CUTOFF)
    inv = 1.0 / x
    x2 = x * x
    x5 = x2 * x2 * x
    envl = inv + x5 * (-28.0 + x * (48.0 - 21.0 * x))
    coef = envl * inv * np.float32(np.sqrt(2.0 / _CUTOFF))
    th = np.float32(np.pi) * x
    s1 = jnp.sin(th)
    c2 = 2.0 * jnp.cos(th)
    s2 = c2 * s1
    s3 = c2 * s2 - s1
    s4 = c2 * s3 - s2
    s5 = c2 * s4 - s3
    s6 = c2 * s5 - s4
    env_ref[0, 0] = coef * s1
    env_ref[0, 1] = coef * s2
    env_ref[0, 2] = coef * s3
    env_ref[0, 3] = coef * s4
    env_ref[0, 4] = coef * s5
    env_ref[0, 5] = coef * s6


def kernel(Z, edge_index, rbf, d, emb_table, W_rbf, b_rbf, W_dense, b_dense):
    Zpad = jnp.pad(Z.astype(jnp.int32), (0, _NPAD - _N))
    ei = edge_index.astype(jnp.int32)
    zsrc, zdst, hpad = _sc_gather(ei[0], ei[1],
                                  Z.astype(jnp.int32), Zpad, emb_table)

    zsrc3 = zsrc.reshape(_GRID, _BE, 1)
    zdst3 = zdst.reshape(_GRID, _BE, 1)
    rbf3 = jnp.pad(rbf, ((0, 0), (0, 8 - _R))).reshape(_GRID, _BE, 8)
    d3 = d.reshape(_GRID, _DSUB, 128)
    emb_pad = jnp.pad(emb_table, ((0, _EMB - emb_table.shape[0]), (0, 0)))
    wrbf_pad = jnp.pad(W_rbf, ((0, 8 - _R), (0, 0)))
    brbf2 = b_rbf.reshape(1, _EMB)
    bdense2 = b_dense.reshape(1, _EMB)

    m3, env3 = pl.pallas_call(
        _tc_body,
        grid=(_GRID,),
        in_specs=[
            pl.BlockSpec((1, _BE, 1), lambda i: (i, 0, 0)),
            pl.BlockSpec((1, _BE, 1), lambda i: (i, 0, 0)),
            pl.BlockSpec((1, _BE, 8), lambda i: (i, 0, 0)),
            pl.BlockSpec((1, _DSUB, 128), lambda i: (i, 0, 0)),
            pl.BlockSpec((_EMB, _EMB), lambda i: (0, 0)),
            pl.BlockSpec((3 * _EMB, _EMB), lambda i: (0, 0)),
            pl.BlockSpec((8, _EMB), lambda i: (0, 0)),
            pl.BlockSpec((1, _EMB), lambda i: (0, 0)),
            pl.BlockSpec((1, _EMB), lambda i: (0, 0)),
        ],
        out_specs=[
            pl.BlockSpec((1, _BE, _EMB), lambda i: (i, 0, 0)),
            pl.BlockSpec((1, _R, _DSUB, 128), lambda i: (i, 0, 0, 0)),
        ],
        out_shape=[
            jax.ShapeDtypeStruct((_GRID, _BE, _EMB), jnp.float32),
            jax.ShapeDtypeStruct((_GRID, _R, _DSUB, 128), jnp.float32),
        ],
        scratch_shapes=[
            pltpu.VMEM((_EMB, _EMB), jnp.float32),
            pltpu.VMEM((_EMB, _EMB), jnp.float32),
            pltpu.VMEM((8, _EMB), jnp.float32),
            pltpu.VMEM((1, _EMB), jnp.float32),
        ],
    )(zsrc3, zdst3, rbf3, d3, emb_pad, W_dense, wrbf_pad, brbf2, bdense2)

    h = hpad[:_N]
    m = m3.reshape(_E, _EMB)
    rbf_env = env3.transpose(0, 2, 3, 1).reshape(_E, _R)
    return (h, m, rbf_env)
